# baseline (speedup 1.0000x reference)
import functools

import jax
import jax.numpy as jnp
from jax import lax
from jax.experimental import pallas as pl
from jax.experimental.pallas import tpu as pltpu
from jax.experimental.pallas import tpu_sc as plsc

BATCH = 1024
IN_SIZE = 100000
FREQ = 90000
HID0 = 256
TAIL = 64
TAILP = 128
HID1 = 128
OUT = 1000
NNZ = 131072

NC = 2
NS = 16
NW = NC * NS
NQ = 4
NR = NW // NQ
QROWS = BATCH // NQ
PER_R = NNZ // NR
STAGE = 512
GB = 32
CAP_F = 1536
CAP_R = 512
PAD_IDX_MASK = 8191


def _i16(v):
    return jnp.full((16,), v, jnp.int32)


def _sc_embed_body(cols_hbm, rows_hbm, vals_hbm, wf_hbm, wr_hbm,
                   hf_out, hr_out,
                   colb, rowb, valb,
                   klf, vlf, klr, vlr,
                   gcf, gcr,
                   fbuf, rbuf, acc_f, acc_r, semf, semr, sstage):
    cid = lax.axis_index("c")
    sid = lax.axis_index("s")
    wid = cid * NS + sid
    rb = wid // NQ
    q = wid % NQ
    qlo = q * QROWS
    lanes = lax.iota(jnp.int32, 16)

    zf = jnp.zeros((16,), jnp.float32)

    def _zero_body(j, _):
        for k in range(HID0 // 16):
            acc_f[j, pl.ds(16 * k, 16)] = zf
        for k in range(TAIL // 16):
            acc_r[j, pl.ds(16 * k, 16)] = zf
        return _
    lax.fori_loop(0, QROWS, _zero_body, None)

    def _process(klist, vlist, gcol, table, buf, acc, sem, nv, cnt, pad):
        if pad:
            pk = (lanes & PAD_IDX_MASK) * 256
            for k in range(GB // 16):
                sl = pl.ds(cnt + 16 * k, 16)
                klist[sl] = pk + 16 * k * 256
                vlist[sl] = zf
            nb = (cnt + GB - 1) // GB
        else:
            nb = cnt // GB

        def _batch(b, _):
            slot = b & 1

            @pl.when(b < nb)
            def _():
                base = b * GB
                for k in range(GB // 16):
                    sl = pl.ds(base + 16 * k, 16)
                    gcol[slot, pl.ds(16 * k, 16)] = klist[sl] >> 8
                pltpu.async_copy(table.at[gcol.at[slot]], buf.at[slot],
                                 sem.at[slot])

            @pl.when(b > 0)
            def _():
                pslot = (b - 1) & 1
                pbase = (b - 1) * GB
                pltpu.make_async_copy(table.at[gcol.at[pslot]],
                                      buf.at[pslot], sem.at[pslot]).wait()

                @plsc.parallel_loop(0, GB // 16, unroll=2)
                def _acc(g):
                    k16 = klist[pl.ds(pbase + 16 * g, 16)]
                    rl16 = k16 & 255
                    vv16 = vlist[pl.ds(pbase + 16 * g, 16)]
                    for t in range(16):
                        j = 16 * g + t
                        rl = jnp.full((16,), rl16[t], jnp.int32)
                        vv = jnp.full((16,), vv16[t], jnp.float32)
                        for k in range(nv):
                            x = buf[pslot, j, pl.ds(16 * k, 16)] * vv
                            plsc.addupdate_scatter(acc, [rl, lanes + 16 * k],
                                                   x)
            return _

        lax.fori_loop(0, nb + 1, _batch, None)
        return nb * GB

    def _overflow(klist, vlist, gcol, table, buf, acc, sem, nv, cnt):
        done = _process(klist, vlist, gcol, table, buf, acc, sem, nv, cnt,
                        pad=False)
        for k in range(GB // 16):
            sl = pl.ds(16 * k, 16)
            klist[sl] = klist[pl.ds(done + 16 * k, 16)]
            vlist[sl] = vlist[pl.ds(done + 16 * k, 16)]

    def _issue_stage(si):
        slot = si & 1
        base = rb * PER_R + si * STAGE
        pltpu.async_copy(cols_hbm.at[pl.ds(base, STAGE)], colb.at[slot],
                         sstage.at[slot])
        pltpu.async_copy(rows_hbm.at[pl.ds(base, STAGE)], rowb.at[slot],
                         sstage.at[slot])
        pltpu.async_copy(vals_hbm.at[pl.ds(base, STAGE)], valb.at[slot],
                         sstage.at[slot])

    def _wait_stage(si):
        slot = si & 1
        z = pl.ds(0, STAGE)
        pltpu.make_async_copy(cols_hbm.at[z], colb.at[slot],
                              sstage.at[slot]).wait()
        pltpu.make_async_copy(rows_hbm.at[z], rowb.at[slot],
                              sstage.at[slot]).wait()
        pltpu.make_async_copy(vals_hbm.at[z], valb.at[slot],
                              sstage.at[slot]).wait()

    _issue_stage(0)

    def _stage_body(si, carry):
        slot = si & 1
        _wait_stage(si)

        @pl.when(si + 1 < PER_R // STAGE)
        def _():
            _issue_stage(si + 1)

        def _scan_body(gi, carry):
            cnt_f, cnt_r = carry
            sl = pl.ds(gi * 16, 16)
            c = colb[slot, sl]
            r = rowb[slot, sl]
            v = valb[slot, sl]
            rl = r - qlo
            inq = (r >= qlo) & (r < qlo + QROWS)
            isf = c < FREQ
            mf = inq & isf
            mr = inq & (~isf)
            key = c * 256 + rl
            plsc.store_compressed(klf.at[pl.ds(cnt_f, 16)], key, mask=mf)
            plsc.store_compressed(vlf.at[pl.ds(cnt_f, 16)], v, mask=mf)
            cnt_f = cnt_f + plsc.all_reduce_population_count(mf)[0]
            plsc.store_compressed(klr.at[pl.ds(cnt_r, 16)],
                                  key - FREQ * 256, mask=mr)
            plsc.store_compressed(vlr.at[pl.ds(cnt_r, 16)], v, mask=mr)
            cnt_r = cnt_r + plsc.all_reduce_population_count(mr)[0]

            ovf_f = cnt_f > CAP_F - 16

            @pl.when(ovf_f)
            def _():
                _overflow(klf, vlf, gcf, wf_hbm, fbuf, acc_f, semf,
                          HID0 // 16, cnt_f)

            cnt_f = jnp.where(ovf_f, cnt_f % GB, cnt_f)

            ovf_r = cnt_r > CAP_R - 16

            @pl.when(ovf_r)
            def _():
                _overflow(klr, vlr, gcr, wr_hbm, rbuf, acc_r, semr,
                          TAIL // 16, cnt_r)

            cnt_r = jnp.where(ovf_r, cnt_r % GB, cnt_r)
            return (cnt_f, cnt_r)

        return lax.fori_loop(0, STAGE // 16, _scan_body, carry)

    cnt_f, cnt_r = lax.fori_loop(0, PER_R // STAGE, _stage_body,
                                 (jnp.int32(0), jnp.int32(0)))

    _process(klf, vlf, gcf, wf_hbm, fbuf, acc_f, semf, HID0 // 16, cnt_f,
             pad=True)
    _process(klr, vlr, gcr, wr_hbm, rbuf, acc_r, semr, TAIL // 16, cnt_r,
             pad=True)

    pos = (q * NR + rb) * QROWS
    pltpu.sync_copy(acc_f, hf_out.at[pl.ds(pos, QROWS)])
    pltpu.sync_copy(acc_r, hr_out.at[pl.ds(pos, QROWS)])


_sc_embed = functools.partial(
    pl.kernel,
    out_type=(
        jax.ShapeDtypeStruct((NW * QROWS, HID0), jnp.float32),
        jax.ShapeDtypeStruct((NW * QROWS, TAIL), jnp.float32),
    ),
    mesh=plsc.VectorSubcoreMesh(core_axis_name="c", subcore_axis_name="s"),
    compiler_params=pltpu.CompilerParams(needs_layout_passes=False),
    scratch_types=(
        pltpu.VMEM((2, STAGE), jnp.int32),
        pltpu.VMEM((2, STAGE), jnp.int32),
        pltpu.VMEM((2, STAGE), jnp.float32),
        pltpu.VMEM((CAP_F + 2 * GB,), jnp.int32),
        pltpu.VMEM((CAP_F + 2 * GB,), jnp.float32),
        pltpu.VMEM((CAP_R + 2 * GB,), jnp.int32),
        pltpu.VMEM((CAP_R + 2 * GB,), jnp.float32),
        pltpu.VMEM((2, GB), jnp.int32),
        pltpu.VMEM((2, GB), jnp.int32),
        pltpu.VMEM((2, GB, HID0), jnp.float32),
        pltpu.VMEM((2, GB, TAILP), jnp.float32),
        pltpu.VMEM((QROWS, HID0), jnp.float32),
        pltpu.VMEM((QROWS, TAIL), jnp.float32),
        pltpu.SemaphoreType.DMA((2,)),
        pltpu.SemaphoreType.DMA((2,)),
        pltpu.SemaphoreType.DMA((2,)),
    ),
)(_sc_embed_body)


def _dense_body(hfp, hrp, bf, br1, wr2, wm, bm, wl, bl, out_ref):
    hf_qs = []
    hr_qs = []
    for q in range(NQ):
        hf_q = hfp[pl.ds(q * NR * QROWS, QROWS), :]
        hr_q = hrp[pl.ds(q * NR * QROWS, QROWS), :]
        for r in range(1, NR):
            hf_q = hf_q + hfp[pl.ds((q * NR + r) * QROWS, QROWS), :]
            hr_q = hr_q + hrp[pl.ds((q * NR + r) * QROWS, QROWS), :]
        hf_qs.append(hf_q)
        hr_qs.append(hr_q)
    hf = jnp.concatenate(hf_qs, axis=0)
    hr = jnp.concatenate(hr_qs, axis=0) + br1[0:1, :]
    h0 = hf + bf[0:1, :] + lax.dot(
        hr, wr2[...], precision=lax.Precision.HIGHEST,
        preferred_element_type=jnp.float32)
    h1 = lax.dot(jnp.maximum(h0, 0.0), wm[...],
                 precision=lax.Precision.HIGHEST,
                 preferred_element_type=jnp.float32) + bm[0:1, :]
    out_ref[...] = lax.dot(jnp.maximum(h1, 0.0), wl[...],
                           precision=lax.Precision.HIGHEST,
                           preferred_element_type=jnp.float32) + bl[0:1, :]


_dense_call = pl.pallas_call(
    _dense_body,
    out_shape=jax.ShapeDtypeStruct((BATCH, OUT), jnp.float32),
)


def kernel(x_indices, x_values, W_freq, b_freq, W_rare1, b_rare1, W_rare2,
           W_mid, b_mid, W_last, b_last):
    rows = x_indices[0]
    cols = x_indices[1]
    W_rare1p = jnp.pad(W_rare1, ((0, 0), (0, TAILP - TAIL)))
    hf2, hr2 = _sc_embed(cols, rows, x_values, W_freq, W_rare1p)
    return _dense_call(
        hf2, hr2,
        b_freq.reshape(1, HID0), b_rare1.reshape(1, TAIL), W_rare2,
        W_mid, b_mid.reshape(1, HID1), W_last, b_last.reshape(1, OUT))


# speedup vs baseline: 2.8409x; 2.8409x over previous
import functools

import jax
import jax.numpy as jnp
from jax import lax
from jax.experimental import pallas as pl
from jax.experimental.pallas import tpu as pltpu
from jax.experimental.pallas import tpu_sc as plsc

BATCH = 1024
IN_SIZE = 100000
FREQ = 90000
HID0 = 256
TAIL = 64
TAILP = 128
HID1 = 128
OUT = 1000
NNZ = 131072

NC = 2
NS = 16
NW = NC * NS
NQ = 4
NR = NW // NQ
QROWS = BATCH // NQ
PER_R = NNZ // NR
STAGE = 512
GB = 32
CAP_F = 1536
CAP_R = 512
PAD_IDX_MASK = 8191


def _i16(v):
    return jnp.full((16,), v, jnp.int32)


def _sc_embed_body(xind_hbm, vals_hbm, wf_hbm, wr_hbm,
                   hf_out, hr_out,
                   colb, rowb, valb,
                   klf, vlf, klr, vlr,
                   gcf, gcr,
                   fbuf, rbuf, acc_f, acc_r, semf, semr, sstage):
    cid = lax.axis_index("c")
    sid = lax.axis_index("s")
    wid = cid * NS + sid
    rb = wid // NQ
    q = wid % NQ
    qlo = q * QROWS
    lanes = lax.iota(jnp.int32, 16)

    zf = jnp.zeros((16,), jnp.float32)

    def _zero_body(j, _):
        for k in range(HID0 // 16):
            acc_f[j, pl.ds(16 * k, 16)] = zf
        for k in range(TAIL // 16):
            acc_r[j, pl.ds(16 * k, 16)] = zf
        return _
    lax.fori_loop(0, QROWS, _zero_body, None)

    def _process(klist, vlist, gcol, table, buf, acc, sem, nv, cnt, pad):
        if pad:
            pk = (lanes & PAD_IDX_MASK) * 256
            for k in range(GB // 16):
                sl = pl.ds(cnt + 16 * k, 16)
                klist[sl] = pk + 16 * k * 256
                vlist[sl] = zf
            nb = (cnt + GB - 1) // GB
        else:
            nb = cnt // GB

        def _batch(b, _):
            slot = b & 1

            @pl.when(b < nb)
            def _():
                base = b * GB
                for k in range(GB // 16):
                    sl = pl.ds(base + 16 * k, 16)
                    gcol[slot, pl.ds(16 * k, 16)] = klist[sl] >> 8
                pltpu.async_copy(table.at[gcol.at[slot]], buf.at[slot],
                                 sem.at[slot])

            @pl.when(b > 0)
            def _():
                pslot = (b - 1) & 1
                pbase = (b - 1) * GB
                pltpu.make_async_copy(table.at[gcol.at[pslot]],
                                      buf.at[pslot], sem.at[pslot]).wait()

                @plsc.parallel_loop(0, GB, unroll=4)
                def _acc(j):
                    kk = plsc.load_gather(klist, [_i16(pbase + j)])
                    rl = kk & 255
                    vv = plsc.load_gather(vlist, [_i16(pbase + j)])
                    for k in range(nv):
                        x = buf[pslot, j, pl.ds(16 * k, 16)] * vv
                        plsc.addupdate_scatter(acc, [rl, lanes + 16 * k], x)
            return _

        lax.fori_loop(0, nb + 1, _batch, None)
        return nb * GB

    def _overflow(klist, vlist, gcol, table, buf, acc, sem, nv, cnt):
        done = _process(klist, vlist, gcol, table, buf, acc, sem, nv, cnt,
                        pad=False)
        for k in range(GB // 16):
            sl = pl.ds(16 * k, 16)
            klist[sl] = klist[pl.ds(done + 16 * k, 16)]
            vlist[sl] = vlist[pl.ds(done + 16 * k, 16)]

    def _issue_stage(si):
        slot = si & 1
        base = rb * PER_R + si * STAGE
        pltpu.async_copy(xind_hbm.at[1, pl.ds(base, STAGE)], colb.at[slot],
                         sstage.at[slot])
        pltpu.async_copy(xind_hbm.at[0, pl.ds(base, STAGE)], rowb.at[slot],
                         sstage.at[slot])
        pltpu.async_copy(vals_hbm.at[pl.ds(base, STAGE)], valb.at[slot],
                         sstage.at[slot])

    def _wait_stage(si):
        slot = si & 1
        z = pl.ds(0, STAGE)
        pltpu.make_async_copy(xind_hbm.at[1, z], colb.at[slot],
                              sstage.at[slot]).wait()
        pltpu.make_async_copy(xind_hbm.at[0, z], rowb.at[slot],
                              sstage.at[slot]).wait()
        pltpu.make_async_copy(vals_hbm.at[z], valb.at[slot],
                              sstage.at[slot]).wait()

    _issue_stage(0)

    def _stage_body(si, carry):
        slot = si & 1
        _wait_stage(si)

        @pl.when(si + 1 < PER_R // STAGE)
        def _():
            _issue_stage(si + 1)

        def _scan_body(gi, carry):
            cnt_f, cnt_r = carry
            sl = pl.ds(gi * 16, 16)
            c = colb[slot, sl]
            r = rowb[slot, sl]
            v = valb[slot, sl]
            rl = r - qlo
            inq = (r >= qlo) & (r < qlo + QROWS)
            isf = c < FREQ
            mf = inq & isf
            mr = inq & (~isf)
            key = c * 256 + rl
            plsc.store_compressed(klf.at[pl.ds(cnt_f, 16)], key, mask=mf)
            plsc.store_compressed(vlf.at[pl.ds(cnt_f, 16)], v, mask=mf)
            cnt_f = cnt_f + plsc.all_reduce_population_count(mf)[0]
            plsc.store_compressed(klr.at[pl.ds(cnt_r, 16)],
                                  key - FREQ * 256, mask=mr)
            plsc.store_compressed(vlr.at[pl.ds(cnt_r, 16)], v, mask=mr)
            cnt_r = cnt_r + plsc.all_reduce_population_count(mr)[0]

            ovf_f = cnt_f > CAP_F - 16

            @pl.when(ovf_f)
            def _():
                _overflow(klf, vlf, gcf, wf_hbm, fbuf, acc_f, semf,
                          HID0 // 16, cnt_f)

            cnt_f = jnp.where(ovf_f, cnt_f % GB, cnt_f)

            ovf_r = cnt_r > CAP_R - 16

            @pl.when(ovf_r)
            def _():
                _overflow(klr, vlr, gcr, wr_hbm, rbuf, acc_r, semr,
                          TAIL // 16, cnt_r)

            cnt_r = jnp.where(ovf_r, cnt_r % GB, cnt_r)
            return (cnt_f, cnt_r)

        return lax.fori_loop(0, STAGE // 16, _scan_body, carry)

    cnt_f, cnt_r = lax.fori_loop(0, PER_R // STAGE, _stage_body,
                                 (jnp.int32(0), jnp.int32(0)))

    _process(klf, vlf, gcf, wf_hbm, fbuf, acc_f, semf, HID0 // 16, cnt_f,
             pad=True)
    _process(klr, vlr, gcr, wr_hbm, rbuf, acc_r, semr, TAIL // 16, cnt_r,
             pad=True)

    pos = (q * NR + rb) * QROWS
    pltpu.sync_copy(acc_f, hf_out.at[pl.ds(pos, QROWS)])
    pltpu.sync_copy(acc_r, hr_out.at[pl.ds(pos, QROWS)])


_sc_embed = functools.partial(
    pl.kernel,
    out_type=(
        jax.ShapeDtypeStruct((NW * QROWS, HID0), jnp.float32),
        jax.ShapeDtypeStruct((NW * QROWS, TAIL), jnp.float32),
    ),
    mesh=plsc.VectorSubcoreMesh(core_axis_name="c", subcore_axis_name="s"),
    compiler_params=pltpu.CompilerParams(needs_layout_passes=False),
    scratch_types=(
        pltpu.VMEM((2, STAGE), jnp.int32),
        pltpu.VMEM((2, STAGE), jnp.int32),
        pltpu.VMEM((2, STAGE), jnp.float32),
        pltpu.VMEM((CAP_F + 2 * GB,), jnp.int32),
        pltpu.VMEM((CAP_F + 2 * GB,), jnp.float32),
        pltpu.VMEM((CAP_R + 2 * GB,), jnp.int32),
        pltpu.VMEM((CAP_R + 2 * GB,), jnp.float32),
        pltpu.VMEM((2, GB), jnp.int32),
        pltpu.VMEM((2, GB), jnp.int32),
        pltpu.VMEM((2, GB, HID0), jnp.float32),
        pltpu.VMEM((2, GB, TAILP), jnp.float32),
        pltpu.VMEM((QROWS, HID0), jnp.float32),
        pltpu.VMEM((QROWS, TAIL), jnp.float32),
        pltpu.SemaphoreType.DMA((2,)),
        pltpu.SemaphoreType.DMA((2,)),
        pltpu.SemaphoreType.DMA((2,)),
    ),
)(_sc_embed_body)


def _dense_body(hfp, hrp, bf, br1, wr2, wm, bm, wl, bl, out_ref):
    hf_qs = []
    hr_qs = []
    for q in range(NQ):
        hf_q = hfp[pl.ds(q * NR * QROWS, QROWS), :]
        hr_q = hrp[pl.ds(q * NR * QROWS, QROWS), :]
        for r in range(1, NR):
            hf_q = hf_q + hfp[pl.ds((q * NR + r) * QROWS, QROWS), :]
            hr_q = hr_q + hrp[pl.ds((q * NR + r) * QROWS, QROWS), :]
        hf_qs.append(hf_q)
        hr_qs.append(hr_q)
    hf = jnp.concatenate(hf_qs, axis=0)
    hr = jnp.concatenate(hr_qs, axis=0) + br1[0:1, :]
    h0 = hf + bf[0:1, :] + lax.dot(
        hr, wr2[...], precision=lax.Precision.HIGHEST,
        preferred_element_type=jnp.float32)
    h1 = lax.dot(jnp.maximum(h0, 0.0), wm[...],
                 precision=lax.Precision.HIGHEST,
                 preferred_element_type=jnp.float32) + bm[0:1, :]
    out_ref[...] = lax.dot(jnp.maximum(h1, 0.0), wl[...],
                           precision=lax.Precision.HIGHEST,
                           preferred_element_type=jnp.float32) + bl[0:1, :]


_dense_call = pl.pallas_call(
    _dense_body,
    out_shape=jax.ShapeDtypeStruct((BATCH, OUT), jnp.float32),
)


def kernel(x_indices, x_values, W_freq, b_freq, W_rare1, b_rare1, W_rare2,
           W_mid, b_mid, W_last, b_last):
    W_rare1p = jnp.pad(W_rare1, ((0, 0), (0, TAILP - TAIL)))
    hf2, hr2 = _sc_embed(x_indices, x_values, W_freq, W_rare1p)
    return _dense_call(
        hf2, hr2,
        b_freq.reshape(1, HID0), b_rare1.reshape(1, TAIL), W_rare2,
        W_mid, b_mid.reshape(1, HID1), W_last, b_last.reshape(1, OUT))
